# baseline (speedup 1.0000x reference)
import functools

import jax
import jax.numpy as jnp
from jax import lax
from jax.experimental import pallas as pl
from jax.experimental.pallas import tpu as pltpu
from jax.experimental.pallas import tpu_sc as plsc

B = 16384
T = 16
NC = 2
NS = 16
NW = NC * NS
BPW = B // NW
CH = 128
NCH = BPW // CH
UNROLL = 8

_C = (5.621959008883515e-07, 0.9999574869, -0.4992065690, 0.3269731000,
      -0.2228362580, 0.1307650330, -0.0526248514, 0.0101190829)


def _softplus(x):
    m = jnp.maximum(x, 0.0)
    t = jnp.exp(-jnp.abs(x))
    p = jnp.full((16,), _C[7], dtype=jnp.float32)
    for k in range(6, -1, -1):
        p = p * t + _C[k]
    return m + p


def _sc_body(loss_hbm, idx_hbm, table_hbm, out_hbm, idx_v, rows_v, loss_v,
             out_v, gsem, lsem):
    wid = lax.axis_index("s") * NC + lax.axis_index("c")

    pltpu.sync_copy(idx_hbm.at[wid], idx_v)
    loss_cp = pltpu.async_copy(loss_hbm.at[wid], loss_v, lsem)
    gathers = [
        pltpu.async_copy(table_hbm.at[idx_v.at[j]],
                         rows_v.at[pl.ds(j * CH, CH)], gsem)
        for j in range(NCH)
    ]
    loss_cp.wait()
    for cp in gathers:
        cp.wait()

    def body(i, acc):
        base = i * UNROLL
        for u in range(UNROLL):
            x = rows_v[base + u]
            l = loss_v[base + u]
            acc = acc + _softplus(x) * l
        return acc

    acc = lax.fori_loop(0, BPW // UNROLL, body,
                        jnp.zeros((16,), dtype=jnp.float32))
    out_v[...] = acc
    pltpu.sync_copy(out_v, out_hbm.at[wid])


@jax.jit
def _run(losses_r, idx_r, table):
    mesh = plsc.VectorSubcoreMesh(core_axis_name="c", subcore_axis_name="s")
    f = functools.partial(
        pl.kernel,
        mesh=mesh,
        compiler_params=pltpu.CompilerParams(use_tc_tiling_on_sc=False),
        out_type=jax.ShapeDtypeStruct((NW, 16), jnp.float32),
        scratch_types=[
            pltpu.VMEM((NCH, CH), jnp.int32),
            pltpu.VMEM((BPW, T), jnp.float32),
            pltpu.VMEM((BPW, T), jnp.float32),
            pltpu.VMEM((16,), jnp.float32),
            pltpu.SemaphoreType.DMA,
            pltpu.SemaphoreType.DMA,
        ],
    )(_sc_body)
    return f(losses_r, idx_r, table)


def kernel(losses, sample_id, weights_table):
    idx_r = sample_id.astype(jnp.int32).reshape(NW, NCH, CH)
    losses_r = losses.reshape(NW, BPW, T)
    partials = _run(losses_r, idx_r, weights_table)
    return jnp.sum(partials) * (1.0 / B)
